# baseline (speedup 1.0000x reference)
import jax
import jax.numpy as jnp
from jax import lax
from jax.experimental import pallas as pl
from jax.experimental.pallas import tpu as pltpu

B = 16
L = 2048
D = 300
PR = D * B
RPB = 960
CPB = 512
NI = PR // RPB
NJ = L // CPB


def _tc_body(seq_ref, lenbc_ref, out_ref, mask_ref, acc_ref):
    j = pl.program_id(1)

    @pl.when((pl.program_id(0) == 0) & (j == 0))
    def _():
        ln = lenbc_ref[:, 0:1]
        pos = lax.broadcasted_iota(jnp.int32, (RPB, L), 1).astype(jnp.float32)
        mask_ref[...] = jnp.where(pos < ln, 1.0, 0.0)

    y = seq_ref[...] * mask_ref[:, pl.ds(j * CPB, CPB)]
    ones = jnp.ones((CPB, 1), jnp.float32)
    part = jax.lax.dot_general(y, ones, (((1,), (0,)), ((), ())),
                               preferred_element_type=jnp.float32)
    pb = jnp.broadcast_to(part, (RPB, 128))

    @pl.when(j == 0)
    def _():
        acc_ref[...] = pb

    @pl.when(j > 0)
    def _():
        acc_ref[...] = acc_ref[...] + pb

    @pl.when(j == NJ - 1)
    def _():
        out_ref[...] = (acc_ref[...] / lenbc_ref[...]).reshape(1, RPB, 128)


def _mean_tc(seqT, len_bc):
    return pl.pallas_call(
        _tc_body,
        grid=(NI, NJ),
        in_specs=[
            pl.BlockSpec((RPB, CPB), lambda i, j: (i, j)),
            pl.BlockSpec((RPB, 128), lambda i, j: (0, 0)),
        ],
        out_specs=pl.BlockSpec((1, RPB, 128), lambda i, j: (i, 0, 0)),
        out_shape=jax.ShapeDtypeStruct((NI, RPB, 128), jnp.float32),
        scratch_shapes=[pltpu.VMEM((RPB, L), jnp.float32),
                        pltpu.VMEM((RPB, 128), jnp.float32)],
    )(seqT, len_bc)


def kernel(sequences, lengths):
    seqT = sequences.transpose(2, 0, 1).reshape(PR, L)
    lenf = lengths.astype(jnp.float32)
    len_bc = jnp.broadcast_to(
        jnp.tile(lenf, RPB // B)[:, None], (RPB, 128))
    tc = _mean_tc(seqT, len_bc)
    return tc[:, :, 0].reshape(D, B).T


# speedup vs baseline: 1.4456x; 1.4456x over previous
import jax
import jax.numpy as jnp
from jax import lax
from jax.experimental import pallas as pl
from jax.experimental.pallas import tpu as pltpu

B = 16
L = 2048
D = 300
PR = D * B
RPB = 480
NI = PR // (2 * RPB)


def _tc_body(seq_a, seq_b, lenbc_ref, out_a, out_b, mask_ref):
    @pl.when(pl.program_id(0) == 0)
    def _():
        ln = lenbc_ref[:, 0:1]
        pos = lax.broadcasted_iota(jnp.int32, (RPB, L), 1).astype(jnp.float32)
        mask_ref[...] = jnp.where(pos < ln, 1.0, 0.0)

    ones = jnp.ones((L, 1), jnp.float32)
    m = mask_ref[...]
    for ref, oref in ((seq_a, out_a), (seq_b, out_b)):
        y = ref[...] * m
        res = jax.lax.dot_general(y, ones, (((1,), (0,)), ((), ())),
                                  preferred_element_type=jnp.float32)
        res = res / lenbc_ref[:, 0:1]
        oref[...] = jnp.broadcast_to(res, (RPB, 128)).reshape(1, RPB, 128)


def _mean_tc(seqT, len_bc):
    return pl.pallas_call(
        _tc_body,
        grid=(NI,),
        in_specs=[
            pl.BlockSpec((RPB, L), lambda i: (i, 0)),
            pl.BlockSpec((RPB, L), lambda i: (i + NI, 0)),
            pl.BlockSpec((RPB, 128), lambda i: (0, 0)),
        ],
        out_specs=[
            pl.BlockSpec((1, RPB, 128), lambda i: (i, 0, 0)),
            pl.BlockSpec((1, RPB, 128), lambda i: (i, 0, 0)),
        ],
        out_shape=[
            jax.ShapeDtypeStruct((NI, RPB, 128), jnp.float32),
            jax.ShapeDtypeStruct((NI, RPB, 128), jnp.float32),
        ],
        scratch_shapes=[pltpu.VMEM((RPB, L), jnp.float32)],
    )(seqT, seqT, len_bc)


def kernel(sequences, lengths):
    seqT = sequences.transpose(2, 0, 1).reshape(PR, L)
    lenf = lengths.astype(jnp.float32)
    len_bc = jnp.broadcast_to(
        jnp.tile(lenf, RPB // B)[:, None], (RPB, 128))
    ta, tb = _mean_tc(seqT, len_bc)
    phys = jnp.concatenate([ta[:, :, 0].reshape(PR // 2),
                            tb[:, :, 0].reshape(PR // 2)])
    return phys.reshape(D, B).T
